# Initial kernel scaffold; baseline (speedup 1.0000x reference)
import jax
import jax.numpy as jnp
from jax.experimental import pallas as pl


def kernel(x, phi):
    raise NotImplementedError("write your pallas kernel here")


# speedup vs baseline: 4000.0792x; 4000.0792x over previous
import jax
import jax.numpy as jnp
from jax.experimental import pallas as pl


def _mf_block(x_ref, phi_ref, out_ref):
    l = jnp.log(phi_ref[...])
    d = 0.5 * (l[1, :] - l[0, :])
    c = 0.5 * jnp.sum(l)
    xf = x_ref[...].astype(jnp.float32)
    out_ref[0, 0, :] = c + jnp.sum(xf * d[None, :], axis=1)


def kernel(x, phi):
    Ns, N = x.shape
    B = 256
    nb = Ns // B
    phi_t = phi.T
    out = pl.pallas_call(
        _mf_block,
        grid=(nb,),
        in_specs=[
            pl.BlockSpec((B, N), lambda i: (i, 0)),
            pl.BlockSpec((2, N), lambda i: (0, 0)),
        ],
        out_specs=pl.BlockSpec((1, 1, B), lambda i: (i, 0, 0)),
        out_shape=jax.ShapeDtypeStruct((nb, 1, B), jnp.float32),
    )(x, phi_t)
    return out.reshape(Ns)
